# Initial kernel scaffold; baseline (speedup 1.0000x reference)
import functools

import jax
import jax.numpy as jnp
from jax import lax
from jax.experimental import pallas as pl
from jax.experimental.pallas import tpu as pltpu
from jax.experimental.pallas import tpu_sc as plsc

N = 10000
E = 320000
D = 128
H = 8
C = 16
HC = H * C
CH = 128
NCHUNK = E // CH
NSUB = 16
ROWS_PER_SUB = N // NSUB
NB = 10
BLK = N // NB


def _stage1_body(x_ref, w_ref, a_ref, xp_ref, alr_ref):
    xp = jnp.dot(x_ref[...], w_ref[...], preferred_element_type=jnp.float32,
                 precision=lax.Precision.HIGHEST)
    xp_ref[...] = xp
    alr_ref[...] = jnp.dot(xp, a_ref[...], preferred_element_type=jnp.float32,
                           precision=lax.Precision.HIGHEST)


def _stage1(x, W, A):
    return pl.pallas_call(
        _stage1_body,
        grid=(NB,),
        in_specs=[pl.BlockSpec((BLK, D), lambda i: (i, 0)),
                  pl.BlockSpec((D, HC), lambda i: (0, 0)),
                  pl.BlockSpec((D, 2 * H), lambda i: (0, 0))],
        out_specs=[pl.BlockSpec((BLK, HC), lambda i: (i, 0)),
                   pl.BlockSpec((BLK, 2 * H), lambda i: (i, 0))],
        out_shape=[jax.ShapeDtypeStruct((N, HC), jnp.float32),
                   jax.ShapeDtypeStruct((N, 2 * H), jnp.float32)],
    )(x, W, A)


def _stage3_body(o0_ref, o1_ref, x_ref, wr_ref, out_ref):
    v = o0_ref[...] + o1_ref[...]
    v = jnp.where(v > 0, v, jnp.exp(jnp.minimum(v, 0.0)) - 1.0)
    out_ref[...] = v + jnp.dot(x_ref[...], wr_ref[...],
                               preferred_element_type=jnp.float32,
                               precision=lax.Precision.HIGHEST)


def _stage3(o0, o1, x, W_res):
    return pl.pallas_call(
        _stage3_body,
        grid=(NB,),
        in_specs=[pl.BlockSpec((BLK, HC), lambda i: (i, 0)),
                  pl.BlockSpec((BLK, HC), lambda i: (i, 0)),
                  pl.BlockSpec((BLK, D), lambda i: (i, 0)),
                  pl.BlockSpec((D, HC), lambda i: (0, 0))],
        out_specs=pl.BlockSpec((BLK, HC), lambda i: (i, 0)),
        out_shape=jax.ShapeDtypeStruct((N, HC), jnp.float32),
    )(o0, o1, x, W_res)


_sc_mesh = plsc.VectorSubcoreMesh(core_axis_name="c", subcore_axis_name="s")


@functools.partial(
    pl.kernel,
    out_type=jax.ShapeDtypeStruct((2, N, HC), jnp.float32),
    mesh=_sc_mesh,
    scratch_types=[
        pltpu.VMEM((CH,), jnp.int32),
        pltpu.VMEM((CH,), jnp.int32),
        pltpu.VMEM((CH,), jnp.float32),
        pltpu.VMEM((CH, 16), jnp.float32),
        pltpu.VMEM((CH, 16), jnp.float32),
        pltpu.VMEM((CH, 16), jnp.float32),
        pltpu.VMEM((CH, 16), jnp.float32),
        pltpu.VMEM((CH, HC), jnp.float32),
        pltpu.VMEM((CH, HC), jnp.float32),
        pltpu.VMEM_SHARED((N, 16), jnp.float32),
        pltpu.VMEM_SHARED((N, HC), jnp.float32),
        pltpu.SemaphoreType.DMA,
        pltpu.SemaphoreType.DMA,
        pltpu.SemaphoreType.DMA,
    ],
)
def _sc_edge(alr_hbm, xp_hbm, src_hbm, dst_hbm, ew_hbm, out_hbm,
             src_v, dst_v, ew_v, srows, drows, exco, denr, xrows, contrib,
             den_sh, out_sh, sem0, sem1, sem2):
    core = lax.axis_index("c")
    sub = lax.axis_index("s")
    wid = sub * 2 + core
    lane = lax.iota(jnp.int32, 16)
    z16 = jnp.zeros((16,), jnp.float32)

    def _zero_row(i, _):
        for j in range(HC // 16):
            contrib[i, pl.ds(16 * j, 16)] = z16
        exco[i, :] = z16
        return 0

    lax.fori_loop(0, CH, _zero_row, 0)
    r0 = sub * ROWS_PER_SUB
    for k in range(5):
        pltpu.sync_copy(exco.at[pl.ds(0, 125)],
                        den_sh.at[pl.ds(r0 + k * 125, 125)])
        pltpu.sync_copy(contrib.at[pl.ds(0, 125)],
                        out_sh.at[pl.ds(r0 + k * 125, 125)])
    plsc.subcore_barrier()

    def _load_chunk(c):
        pltpu.sync_copy(src_hbm.at[c], src_v)
        pltpu.sync_copy(dst_hbm.at[c], dst_v)
        pltpu.sync_copy(ew_hbm.at[c], ew_v)
        cp0 = pltpu.async_copy(alr_hbm.at[src_v], srows, sem0)
        cp1 = pltpu.async_copy(alr_hbm.at[dst_v], drows, sem1)
        return cp0, cp1

    def _edge_scores(g):
        e_ids = g * 16 + lane
        ewv = ew_v[pl.ds(g * 16, 16)]
        res = []
        for h in range(H):
            al = plsc.load_gather(srows, [e_ids, jnp.full((16,), h, jnp.int32)])
            ar = plsc.load_gather(drows, [e_ids, jnp.full((16,), h + 8, jnp.int32)])
            t = ewv * (al + ar)
            t = jnp.where(t >= 0, t, 0.2 * t)
            res.append((e_ids, jnp.exp(t)))
        return res

    def _p1(i, _):
        c = sub + i * NSUB
        cp0, cp1 = _load_chunk(c)
        cp0.wait()
        cp1.wait()

        def grp(g, _):
            for h, (e_ids, ex) in enumerate(_edge_scores(g)):
                plsc.store_scatter(exco, [e_ids, jnp.full((16,), h, jnp.int32)], ex)
            return 0

        lax.fori_loop(0, CH // 16, grp, 0)
        pltpu.sync_copy(exco, den_sh.at[dst_v], add=True)
        return 0

    n1 = (NCHUNK - sub + NSUB - 1) // NSUB
    lax.fori_loop(0, n1, _p1, 0)
    plsc.subcore_barrier()

    def _p2(i, _):
        c = wid + i * 32
        cp0, cp1 = _load_chunk(c)
        cp2 = pltpu.async_copy(xp_hbm.at[src_v], xrows, sem2)
        pltpu.sync_copy(den_sh.at[dst_v], denr)
        cp0.wait()
        cp1.wait()

        def grp(g, _):
            for h, (e_ids, ex) in enumerate(_edge_scores(g)):
                den = plsc.load_gather(denr, [e_ids, jnp.full((16,), h, jnp.int32)])
                coeff = ex / (den + 1e-16)
                plsc.store_scatter(exco, [e_ids, jnp.full((16,), h, jnp.int32)], coeff)
            return 0

        lax.fori_loop(0, CH // 16, grp, 0)
        cp2.wait()

        def edge(e, _):
            e16 = jnp.full((16,), e, jnp.int32)
            for h in range(H):
                cv = plsc.load_gather(exco, [e16, jnp.full((16,), h, jnp.int32)])
                contrib[e, pl.ds(h * 16, 16)] = xrows[e, pl.ds(h * 16, 16)] * cv
            return 0

        lax.fori_loop(0, CH, edge, 0)
        pltpu.sync_copy(contrib, out_sh.at[dst_v], add=True)
        return 0

    n2 = (NCHUNK - wid + 31) // 32
    lax.fori_loop(0, n2, _p2, 0)
    plsc.subcore_barrier()

    pltpu.sync_copy(out_sh.at[pl.ds(r0, ROWS_PER_SUB)],
                    out_hbm.at[core, pl.ds(r0, ROWS_PER_SUB)])


def kernel(x, edge_weight, W, att_l, att_r, W_res, edge_index):
    attl = att_l.reshape(H, C)
    attr_ = att_r.reshape(H, C)
    eyeH = jnp.eye(H, dtype=jnp.float32)
    A_l = (attl[:, :, None] * eyeH[:, None, :]).reshape(HC, H)
    A_r = (attr_[:, :, None] * eyeH[:, None, :]).reshape(HC, H)
    A = jnp.concatenate([A_l, A_r], axis=1)

    src2 = edge_index[0].reshape(NCHUNK, CH)
    dst2 = edge_index[1].reshape(NCHUNK, CH)
    ew2 = edge_weight.reshape(NCHUNK, CH)

    xp, alr = _stage1(x, W, A)
    out2 = _sc_edge(alr, xp, src2, dst2, ew2)
    return _stage3(out2[0], out2[1], x, W_res)


# speedup vs baseline: 32.5891x; 32.5891x over previous
import functools

import jax
import jax.numpy as jnp
from jax import lax
from jax.experimental import pallas as pl
from jax.experimental.pallas import tpu as pltpu
from jax.experimental.pallas import tpu_sc as plsc

N = 10000
E = 320000
D = 128
H = 8
C = 16
HC = H * C
CH = 128
NCHUNK = E // CH
NSUB = 16
ROWS_PER_SUB = N // NSUB
NB = 10
BLK = N // NB


def _stage1_body(x_ref, w_ref, a_ref, xp_ref, alr_ref):
    xp = jnp.dot(x_ref[...], w_ref[...], preferred_element_type=jnp.float32,
                 precision=lax.Precision.HIGHEST)
    xp_ref[...] = xp
    alr_ref[...] = jnp.dot(xp, a_ref[...], preferred_element_type=jnp.float32,
                           precision=lax.Precision.HIGHEST)


def _stage1(x, W, A):
    return pl.pallas_call(
        _stage1_body,
        grid=(NB,),
        in_specs=[pl.BlockSpec((BLK, D), lambda i: (i, 0)),
                  pl.BlockSpec((D, HC), lambda i: (0, 0)),
                  pl.BlockSpec((D, 2 * H), lambda i: (0, 0))],
        out_specs=[pl.BlockSpec((BLK, HC), lambda i: (i, 0)),
                   pl.BlockSpec((BLK, 2 * H), lambda i: (i, 0))],
        out_shape=[jax.ShapeDtypeStruct((N, HC), jnp.float32),
                   jax.ShapeDtypeStruct((N, 2 * H), jnp.float32)],
    )(x, W, A)


def _stage3_body(o0_ref, o1_ref, x_ref, wr_ref, out_ref):
    v = o0_ref[...] + o1_ref[...]
    v = jnp.where(v > 0, v, jnp.exp(jnp.minimum(v, 0.0)) - 1.0)
    out_ref[...] = v + jnp.dot(x_ref[...], wr_ref[...],
                               preferred_element_type=jnp.float32,
                               precision=lax.Precision.HIGHEST)


def _stage3(o0, o1, x, W_res):
    return pl.pallas_call(
        _stage3_body,
        grid=(NB,),
        in_specs=[pl.BlockSpec((BLK, HC), lambda i: (i, 0)),
                  pl.BlockSpec((BLK, HC), lambda i: (i, 0)),
                  pl.BlockSpec((BLK, D), lambda i: (i, 0)),
                  pl.BlockSpec((D, HC), lambda i: (0, 0))],
        out_specs=pl.BlockSpec((BLK, HC), lambda i: (i, 0)),
        out_shape=jax.ShapeDtypeStruct((N, HC), jnp.float32),
    )(o0, o1, x, W_res)


_sc_mesh = plsc.VectorSubcoreMesh(core_axis_name="c", subcore_axis_name="s")


@functools.partial(
    pl.kernel,
    out_type=jax.ShapeDtypeStruct((2, N, HC), jnp.float32),
    mesh=_sc_mesh,
    compiler_params=pltpu.CompilerParams(needs_layout_passes=False,
                                         use_tc_tiling_on_sc=False),
    scratch_types=[
        pltpu.VMEM((CH,), jnp.int32),
        pltpu.VMEM((CH,), jnp.int32),
        pltpu.VMEM((CH,), jnp.float32),
        pltpu.VMEM((CH, 16), jnp.float32),
        pltpu.VMEM((CH, 16), jnp.float32),
        pltpu.VMEM((CH, H), jnp.float32),
        pltpu.VMEM((CH, H), jnp.float32),
        pltpu.VMEM((CH, HC), jnp.float32),
        pltpu.VMEM((CH, HC), jnp.float32),
        pltpu.VMEM_SHARED((N, H), jnp.float32),
        pltpu.VMEM_SHARED((N, HC), jnp.float32),
        pltpu.SemaphoreType.DMA,
        pltpu.SemaphoreType.DMA,
        pltpu.SemaphoreType.DMA,
    ],
)
def _sc_edge(alr_hbm, xp_hbm, src_hbm, dst_hbm, ew_hbm, out_hbm,
             src_v, dst_v, ew_v, srows, drows, exco, denr, xrows, contrib,
             den_sh, out_sh, sem0, sem1, sem2):
    core = lax.axis_index("c")
    sub = lax.axis_index("s")
    wid = sub * 2 + core
    lane = lax.iota(jnp.int32, 16)
    z16 = jnp.zeros((16,), jnp.float32)

    def _zero_row(i, _):
        for j in range(HC // 16):
            contrib[i, pl.ds(16 * j, 16)] = z16
        plsc.store_scatter(exco, [jnp.full((16,), i, jnp.int32),
                                  jnp.bitwise_and(lane, 7)], z16)
        return 0

    lax.fori_loop(0, CH, _zero_row, 0)
    r0 = sub * ROWS_PER_SUB
    for k in range(5):
        pltpu.sync_copy(exco.at[pl.ds(0, 125)],
                        den_sh.at[pl.ds(r0 + k * 125, 125)])
        pltpu.sync_copy(contrib.at[pl.ds(0, 125)],
                        out_sh.at[pl.ds(r0 + k * 125, 125)])
    plsc.subcore_barrier()

    def _load_chunk(c):
        pltpu.sync_copy(src_hbm.at[c], src_v)
        pltpu.sync_copy(dst_hbm.at[c], dst_v)
        pltpu.sync_copy(ew_hbm.at[c], ew_v)
        cp0 = pltpu.async_copy(alr_hbm.at[src_v], srows, sem0)
        cp1 = pltpu.async_copy(alr_hbm.at[dst_v], drows, sem1)
        return cp0, cp1

    def _edge_scores(g):
        e_ids = g * 16 + lane
        ewv = ew_v[pl.ds(g * 16, 16)]
        res = []
        for h in range(H):
            al = plsc.load_gather(srows, [e_ids, jnp.full((16,), h, jnp.int32)])
            ar = plsc.load_gather(drows, [e_ids, jnp.full((16,), h + 8, jnp.int32)])
            t = ewv * (al + ar)
            t = jnp.where(t >= 0, t, 0.2 * t)
            res.append((e_ids, jnp.exp(t)))
        return res

    def _p1(i, _):
        c = sub + i * NSUB
        cp0, cp1 = _load_chunk(c)
        cp0.wait()
        cp1.wait()

        def grp(g, _):
            for h, (e_ids, ex) in enumerate(_edge_scores(g)):
                plsc.store_scatter(exco, [e_ids, jnp.full((16,), h, jnp.int32)], ex)
            return 0

        lax.fori_loop(0, CH // 16, grp, 0)
        pltpu.sync_copy(exco, den_sh.at[dst_v], add=True)
        return 0

    n1 = (NCHUNK - sub + NSUB - 1) // NSUB
    lax.fori_loop(0, n1, _p1, 0)
    plsc.subcore_barrier()

    def _p2(i, _):
        c = wid + i * 32
        cp0, cp1 = _load_chunk(c)
        cp2 = pltpu.async_copy(xp_hbm.at[src_v], xrows, sem2)
        pltpu.sync_copy(den_sh.at[dst_v], denr)
        cp0.wait()
        cp1.wait()

        def grp(g, _):
            for h, (e_ids, ex) in enumerate(_edge_scores(g)):
                den = plsc.load_gather(denr, [e_ids, jnp.full((16,), h, jnp.int32)])
                coeff = ex / (den + 1e-16)
                plsc.store_scatter(exco, [e_ids, jnp.full((16,), h, jnp.int32)], coeff)
            return 0

        lax.fori_loop(0, CH // 16, grp, 0)
        cp2.wait()

        def edge(e, _):
            e16 = jnp.full((16,), e, jnp.int32)
            for h in range(H):
                cv = plsc.load_gather(exco, [e16, jnp.full((16,), h, jnp.int32)])
                contrib[e, pl.ds(h * 16, 16)] = xrows[e, pl.ds(h * 16, 16)] * cv
            return 0

        lax.fori_loop(0, CH, edge, 0)
        pltpu.sync_copy(contrib, out_sh.at[dst_v], add=True)
        return 0

    n2 = (NCHUNK - wid + 31) // 32
    lax.fori_loop(0, n2, _p2, 0)
    plsc.subcore_barrier()

    r0w = sub * 624
    pltpu.sync_copy(out_sh.at[pl.ds(r0w, 624)],
                    out_hbm.at[core, pl.ds(r0w, 624)])

    @pl.when(sub == NSUB - 1)
    def _tail():
        pltpu.sync_copy(out_sh.at[pl.ds(N - 16, 16)],
                        out_hbm.at[core, pl.ds(N - 16, 16)])


def kernel(x, edge_weight, W, att_l, att_r, W_res, edge_index):
    attl = att_l.reshape(H, C)
    attr_ = att_r.reshape(H, C)
    eyeH = jnp.eye(H, dtype=jnp.float32)
    A_l = (attl[:, :, None] * eyeH[:, None, :]).reshape(HC, H)
    A_r = (attr_[:, :, None] * eyeH[:, None, :]).reshape(HC, H)
    A = jnp.concatenate([A_l, A_r], axis=1)

    src2 = edge_index[0].reshape(NCHUNK, CH)
    dst2 = edge_index[1].reshape(NCHUNK, CH)
    ew2 = edge_weight.reshape(NCHUNK, CH)

    xp, alr = _stage1(x, W, A)
    out2 = _sc_edge(alr, xp, src2, dst2, ew2)
    return _stage3(out2[0], out2[1], x, W_res)


# speedup vs baseline: 46.5300x; 1.4278x over previous
import functools

import jax
import jax.numpy as jnp
from jax import lax
from jax.experimental import pallas as pl
from jax.experimental.pallas import tpu as pltpu
from jax.experimental.pallas import tpu_sc as plsc

N = 10000
E = 320000
D = 128
H = 8
C = 16
HC = H * C
CH = 128
NCHUNK = E // CH
NSUB = 16
ROWS_PER_SUB = N // NSUB
NB = 10
BLK = N // NB


def _stage1_body(x_ref, w_ref, a_ref, xp_ref, alr_ref):
    xp = jnp.dot(x_ref[...], w_ref[...], preferred_element_type=jnp.float32,
                 precision=lax.Precision.HIGHEST)
    xp_ref[...] = xp
    alr_ref[...] = jnp.dot(xp, a_ref[...], preferred_element_type=jnp.float32,
                           precision=lax.Precision.HIGHEST)


def _stage1(x, W, A):
    return pl.pallas_call(
        _stage1_body,
        grid=(NB,),
        in_specs=[pl.BlockSpec((BLK, D), lambda i: (i, 0)),
                  pl.BlockSpec((D, HC), lambda i: (0, 0)),
                  pl.BlockSpec((D, 2 * H), lambda i: (0, 0))],
        out_specs=[pl.BlockSpec((BLK, HC), lambda i: (i, 0)),
                   pl.BlockSpec((BLK, 2 * H), lambda i: (i, 0))],
        out_shape=[jax.ShapeDtypeStruct((N, HC), jnp.float32),
                   jax.ShapeDtypeStruct((N, 2 * H), jnp.float32)],
    )(x, W, A)


def _stage3_body(o0_ref, o1_ref, x_ref, wr_ref, out_ref):
    v = o0_ref[...] + o1_ref[...]
    v = jnp.where(v > 0, v, jnp.exp(jnp.minimum(v, 0.0)) - 1.0)
    out_ref[...] = v + jnp.dot(x_ref[...], wr_ref[...],
                               preferred_element_type=jnp.float32,
                               precision=lax.Precision.HIGHEST)


def _stage3(o0, o1, x, W_res):
    return pl.pallas_call(
        _stage3_body,
        grid=(NB,),
        in_specs=[pl.BlockSpec((BLK, HC), lambda i: (i, 0)),
                  pl.BlockSpec((BLK, HC), lambda i: (i, 0)),
                  pl.BlockSpec((BLK, D), lambda i: (i, 0)),
                  pl.BlockSpec((D, HC), lambda i: (0, 0))],
        out_specs=pl.BlockSpec((BLK, HC), lambda i: (i, 0)),
        out_shape=jax.ShapeDtypeStruct((N, HC), jnp.float32),
    )(o0, o1, x, W_res)


_sc_mesh = plsc.VectorSubcoreMesh(core_axis_name="c", subcore_axis_name="s")


P1_ITERS = 158
P2_ITERS = 80


@functools.partial(
    pl.kernel,
    out_type=jax.ShapeDtypeStruct((2, N, HC), jnp.float32),
    mesh=_sc_mesh,
    compiler_params=pltpu.CompilerParams(needs_layout_passes=False,
                                         use_tc_tiling_on_sc=False),
    scratch_types=[
        pltpu.VMEM((3, CH), jnp.int32),
        pltpu.VMEM((3, CH), jnp.int32),
        pltpu.VMEM((CH, 16), jnp.float32),
        pltpu.VMEM((CH, 16), jnp.float32),
        pltpu.VMEM((CH, 16), jnp.float32),
        pltpu.VMEM((CH, 16), jnp.float32),
        pltpu.VMEM((CH, HC), jnp.float32),
        pltpu.VMEM((CH, HC), jnp.float32),
        pltpu.VMEM((CH, H), jnp.float32),
        pltpu.VMEM((CH, H), jnp.float32),
        pltpu.VMEM_SHARED((N, H), jnp.float32),
        pltpu.VMEM_SHARED((N, HC), jnp.float32),
        pltpu.SemaphoreType.DMA,
        pltpu.SemaphoreType.DMA,
        pltpu.SemaphoreType.DMA,
        pltpu.SemaphoreType.DMA,
    ],
)
def _sc_edge(alr_hbm, xp_hbm, pack_hbm, out_hbm,
             pack0, pack1, sr0, sr1, dr0, dr1, xr0, xr1, exco, denr,
             den_sh, out_sh, semi0, semi1, semg0, semg1):
    core = lax.axis_index("c")
    sub = lax.axis_index("s")
    wid = sub * 2 + core
    lane = lax.iota(jnp.int32, 16)
    z16 = jnp.zeros((16,), jnp.float32)

    packs = (pack0, pack1)
    srs = (sr0, sr1)
    drs = (dr0, dr1)
    xrs = (xr0, xr1)
    semis = (semi0, semi1)
    semgs = (semg0, semg1)

    def _zero_row(i, _):
        for j in range(HC // 16):
            xr0[i, pl.ds(16 * j, 16)] = z16
        plsc.store_scatter(exco, [jnp.full((16,), i, jnp.int32),
                                  jnp.bitwise_and(lane, 7)], z16)
        return 0

    lax.fori_loop(0, CH, _zero_row, 0)
    r0 = sub * ROWS_PER_SUB
    for k in range(5):
        pltpu.sync_copy(exco.at[pl.ds(0, 125)],
                        den_sh.at[pl.ds(r0 + k * 125, 125)])
        pltpu.sync_copy(xr0.at[pl.ds(0, 125)],
                        out_sh.at[pl.ds(r0 + k * 125, 125)])
    plsc.subcore_barrier()

    def _edge_scores(pack_b, b, g):
        e_ids = g * 16 + lane
        ewv = plsc.bitcast(pack_b[2, pl.ds(g * 16, 16)], jnp.float32)
        res = []
        for h in range(H):
            al = plsc.load_gather(srs[b], [e_ids, jnp.full((16,), h, jnp.int32)])
            ar = plsc.load_gather(drs[b], [e_ids, jnp.full((16,), h + 8, jnp.int32)])
            t = ewv * (al + ar)
            t = jnp.where(t >= 0, t, 0.2 * t)
            res.append((e_ids, jnp.exp(t)))
        return res

    def _run_pass(chunk_of, iters, with_xp, compute_and_scatter):
        def cidx(it):
            c = chunk_of(it)
            return jnp.minimum(c, NCHUNK - 1), c < NCHUNK

        def fire_idx(it, b):
            c, _ = cidx(it)
            pltpu.async_copy(pack_hbm.at[c], packs[b], semis[b])

        def drain_idx(b):
            pltpu.make_async_copy(pack_hbm.at[0], packs[b], semis[b]).wait()

        def fire_gathers(b):
            pltpu.async_copy(alr_hbm.at[packs[b].at[0]], srs[b], semgs[b])
            pltpu.async_copy(alr_hbm.at[packs[b].at[1]], drs[b], semgs[b])
            if with_xp:
                pltpu.async_copy(xp_hbm.at[packs[b].at[0]], xrs[b], semgs[b])

        def drain_gathers(b):
            pltpu.make_async_copy(alr_hbm.at[packs[b].at[0]], srs[b],
                                  semgs[b]).wait()
            pltpu.make_async_copy(alr_hbm.at[packs[b].at[1]], drs[b],
                                  semgs[b]).wait()
            if with_xp:
                pltpu.make_async_copy(xp_hbm.at[packs[b].at[0]], xrs[b],
                                      semgs[b]).wait()

        c0, _ = cidx(0)
        pltpu.sync_copy(pack_hbm.at[c0], pack0)
        fire_gathers(0)
        fire_idx(1, 1)

        def body(j, _):
            for b in (0, 1):
                it = 2 * j + b
                drain_gathers(b)
                drain_idx(b ^ 1)
                fire_gathers(b ^ 1)
                _, valid = cidx(it)
                compute_and_scatter(b, valid)
                fire_idx(it + 2, b)
            return 0

        lax.fori_loop(0, iters // 2, body, 0)
        drain_gathers(0)
        drain_idx(1)

    def _p1_compute(b, valid):
        def grp(g, _):
            for h, (e_ids, ex) in enumerate(_edge_scores(packs[b], b, g)):
                plsc.store_scatter(exco, [e_ids, jnp.full((16,), h, jnp.int32)],
                                   ex)
            return 0

        lax.fori_loop(0, CH // 16, grp, 0)

        @pl.when(valid)
        def _():
            pltpu.sync_copy(exco, den_sh.at[packs[b].at[1]], add=True)

    _run_pass(lambda it: sub + it * NSUB, P1_ITERS, False, _p1_compute)
    plsc.subcore_barrier()

    def _p2_compute(b, valid):
        pltpu.sync_copy(den_sh.at[packs[b].at[1]], denr)

        def grp(g, _):
            for h, (e_ids, ex) in enumerate(_edge_scores(packs[b], b, g)):
                den = plsc.load_gather(denr,
                                       [e_ids, jnp.full((16,), h, jnp.int32)])
                coeff = ex / (den + 1e-16)
                plsc.store_scatter(exco, [e_ids, jnp.full((16,), h, jnp.int32)],
                                   coeff)
            return 0

        lax.fori_loop(0, CH // 16, grp, 0)

        def edge(e, _):
            e16 = jnp.full((16,), e, jnp.int32)
            for h in range(H):
                cv = plsc.load_gather(exco, [e16, jnp.full((16,), h, jnp.int32)])
                xrs[b][e, pl.ds(h * 16, 16)] = xrs[b][e, pl.ds(h * 16, 16)] * cv
            return 0

        lax.fori_loop(0, CH, edge, 0)

        @pl.when(valid)
        def _():
            pltpu.sync_copy(xrs[b], out_sh.at[packs[b].at[1]], add=True)

    _run_pass(lambda it: wid + it * 32, P2_ITERS, True, _p2_compute)
    plsc.subcore_barrier()

    r0w = sub * 624
    pltpu.sync_copy(out_sh.at[pl.ds(r0w, 624)],
                    out_hbm.at[core, pl.ds(r0w, 624)])

    @pl.when(sub == NSUB - 1)
    def _tail():
        pltpu.sync_copy(out_sh.at[pl.ds(N - 16, 16)],
                        out_hbm.at[core, pl.ds(N - 16, 16)])


def kernel(x, edge_weight, W, att_l, att_r, W_res, edge_index):
    attl = att_l.reshape(H, C)
    attr_ = att_r.reshape(H, C)
    eyeH = jnp.eye(H, dtype=jnp.float32)
    A_l = (attl[:, :, None] * eyeH[:, None, :]).reshape(HC, H)
    A_r = (attr_[:, :, None] * eyeH[:, None, :]).reshape(HC, H)
    A = jnp.concatenate([A_l, A_r], axis=1)

    src2 = edge_index[0].reshape(NCHUNK, CH)
    dst2 = edge_index[1].reshape(NCHUNK, CH)
    ew2 = lax.bitcast_convert_type(edge_weight, jnp.int32).reshape(NCHUNK, CH)
    pack = jnp.stack([src2, dst2, ew2], axis=1)

    xp, alr = _stage1(x, W, A)
    out2 = _sc_edge(alr, xp, pack)
    return _stage3(out2[0], out2[1], x, W_res)


# speedup vs baseline: 78.1740x; 1.6801x over previous
import functools

import jax
import jax.numpy as jnp
from jax import lax
from jax.experimental import pallas as pl
from jax.experimental.pallas import tpu as pltpu
from jax.experimental.pallas import tpu_sc as plsc

N = 10000
E = 320000
D = 128
H = 8
C = 16
HC = H * C
CH = 128
NCHUNK = E // CH
NSUB = 16
ROWS_PER_SUB = N // NSUB
NB = 10
BLK = N // NB


def _stage1_body(x_ref, w_ref, a_ref, xp_ref, alr_ref):
    xp = jnp.dot(x_ref[...], w_ref[...], preferred_element_type=jnp.float32,
                 precision=lax.Precision.HIGHEST)
    xp_ref[...] = xp
    alr_ref[...] = jnp.dot(xp, a_ref[...], preferred_element_type=jnp.float32,
                           precision=lax.Precision.HIGHEST)


def _stage1(x, W, A):
    return pl.pallas_call(
        _stage1_body,
        grid=(NB,),
        in_specs=[pl.BlockSpec((BLK, D), lambda i: (i, 0)),
                  pl.BlockSpec((D, HC), lambda i: (0, 0)),
                  pl.BlockSpec((D, 2 * H), lambda i: (0, 0))],
        out_specs=[pl.BlockSpec((BLK, HC), lambda i: (i, 0)),
                   pl.BlockSpec((BLK, 2 * H), lambda i: (i, 0))],
        out_shape=[jax.ShapeDtypeStruct((N, HC), jnp.float32),
                   jax.ShapeDtypeStruct((N, 2 * H), jnp.float32)],
    )(x, W, A)


def _stage3_body(o0_ref, o1_ref, x_ref, wr_ref, out_ref):
    v = o0_ref[...] + o1_ref[...]
    v = jnp.where(v > 0, v, jnp.exp(jnp.minimum(v, 0.0)) - 1.0)
    out_ref[...] = v + jnp.dot(x_ref[...], wr_ref[...],
                               preferred_element_type=jnp.float32,
                               precision=lax.Precision.HIGHEST)


def _stage3(o0, o1, x, W_res):
    return pl.pallas_call(
        _stage3_body,
        grid=(NB,),
        in_specs=[pl.BlockSpec((BLK, HC), lambda i: (i, 0)),
                  pl.BlockSpec((BLK, HC), lambda i: (i, 0)),
                  pl.BlockSpec((BLK, D), lambda i: (i, 0)),
                  pl.BlockSpec((D, HC), lambda i: (0, 0))],
        out_specs=pl.BlockSpec((BLK, HC), lambda i: (i, 0)),
        out_shape=jax.ShapeDtypeStruct((N, HC), jnp.float32),
    )(o0, o1, x, W_res)


_sc_mesh = plsc.VectorSubcoreMesh(core_axis_name="c", subcore_axis_name="s")


P1_ITERS = 158
P2_ITERS = 80


@functools.partial(
    pl.kernel,
    out_type=jax.ShapeDtypeStruct((2, N, HC), jnp.float32),
    mesh=_sc_mesh,
    compiler_params=pltpu.CompilerParams(needs_layout_passes=False,
                                         use_tc_tiling_on_sc=False),
    scratch_types=[
        pltpu.VMEM((3, CH), jnp.int32),
        pltpu.VMEM((3, CH), jnp.int32),
        pltpu.VMEM((CH, 16), jnp.float32),
        pltpu.VMEM((CH, 16), jnp.float32),
        pltpu.VMEM((CH, 16), jnp.float32),
        pltpu.VMEM((CH, 16), jnp.float32),
        pltpu.VMEM((CH, HC), jnp.float32),
        pltpu.VMEM((CH, HC), jnp.float32),
        pltpu.VMEM((CH, H), jnp.float32),
        pltpu.VMEM((H, CH), jnp.float32),
        pltpu.VMEM((CH, H), jnp.float32),
        pltpu.VMEM((CH, H), jnp.float32),
        pltpu.VMEM_SHARED((N, H), jnp.float32),
        pltpu.VMEM_SHARED((N, HC), jnp.float32),
        pltpu.SemaphoreType.DMA,
        pltpu.SemaphoreType.DMA,
        pltpu.SemaphoreType.DMA,
        pltpu.SemaphoreType.DMA,
    ],
)
def _sc_edge(alr_hbm, xp_hbm, pack_hbm, out_hbm,
             pack0, pack1, sr0, sr1, dr0, dr1, xr0, xr1, exco, cfT, dnr0, dnr1,
             den_sh, out_sh, semi0, semi1, semg0, semg1):
    core = lax.axis_index("c")
    sub = lax.axis_index("s")
    wid = sub * 2 + core
    lane = lax.iota(jnp.int32, 16)
    z16 = jnp.zeros((16,), jnp.float32)

    packs = (pack0, pack1)
    srs = (sr0, sr1)
    drs = (dr0, dr1)
    xrs = (xr0, xr1)
    dnrs = (dnr0, dnr1)
    semis = (semi0, semi1)
    semgs = (semg0, semg1)

    def _zero_row(i, _):
        for j in range(HC // 16):
            xr0[i, pl.ds(16 * j, 16)] = z16
        plsc.store_scatter(exco, [jnp.full((16,), i, jnp.int32),
                                  jnp.bitwise_and(lane, 7)], z16)
        return 0

    lax.fori_loop(0, CH, _zero_row, 0)
    r0 = sub * ROWS_PER_SUB
    for k in range(5):
        pltpu.sync_copy(exco.at[pl.ds(0, 125)],
                        den_sh.at[pl.ds(r0 + k * 125, 125)])
        pltpu.sync_copy(xr0.at[pl.ds(0, 125)],
                        out_sh.at[pl.ds(r0 + k * 125, 125)])
    plsc.subcore_barrier()

    def _edge_scores(pack_b, b, g):
        e_ids = g * 16 + lane
        ewv = plsc.bitcast(pack_b[2, pl.ds(g * 16, 16)], jnp.float32)
        res = []
        for h in range(H):
            al = plsc.load_gather(srs[b], [e_ids, jnp.full((16,), h, jnp.int32)])
            ar = plsc.load_gather(drs[b], [e_ids, jnp.full((16,), h + 8, jnp.int32)])
            t = ewv * (al + ar)
            t = jnp.where(t >= 0, t, 0.2 * t)
            res.append((e_ids, jnp.exp(t)))
        return res

    def _run_pass(chunk_of, iters, with_xp, compute_and_scatter):
        def cidx(it):
            c = chunk_of(it)
            return jnp.minimum(c, NCHUNK - 1), c < NCHUNK

        def fire_idx(it, b):
            c, _ = cidx(it)
            pltpu.async_copy(pack_hbm.at[c], packs[b], semis[b])

        def drain_idx(b):
            pltpu.make_async_copy(pack_hbm.at[0], packs[b], semis[b]).wait()

        def fire_gathers(b):
            pltpu.async_copy(alr_hbm.at[packs[b].at[0]], srs[b], semgs[b])
            pltpu.async_copy(alr_hbm.at[packs[b].at[1]], drs[b], semgs[b])
            if with_xp:
                pltpu.async_copy(xp_hbm.at[packs[b].at[0]], xrs[b], semgs[b])

        def drain_gathers(b):
            pltpu.make_async_copy(alr_hbm.at[packs[b].at[0]], srs[b],
                                  semgs[b]).wait()
            pltpu.make_async_copy(alr_hbm.at[packs[b].at[1]], drs[b],
                                  semgs[b]).wait()
            if with_xp:
                pltpu.make_async_copy(xp_hbm.at[packs[b].at[0]], xrs[b],
                                      semgs[b]).wait()

        c0, _ = cidx(0)
        pltpu.sync_copy(pack_hbm.at[c0], pack0)
        fire_gathers(0)
        fire_idx(1, 1)

        def body(j, _):
            for b in (0, 1):
                it = 2 * j + b
                drain_gathers(b)
                drain_idx(b ^ 1)
                fire_gathers(b ^ 1)
                _, valid = cidx(it)
                compute_and_scatter(b, valid)
                fire_idx(it + 2, b)
            return 0

        lax.fori_loop(0, iters // 2, body, 0)
        drain_gathers(0)
        drain_idx(1)

    def _p1_compute(b, valid):
        def grp(g, _):
            for h, (e_ids, ex) in enumerate(_edge_scores(packs[b], b, g)):
                plsc.store_scatter(exco, [e_ids, jnp.full((16,), h, jnp.int32)],
                                   ex)
            return 0

        lax.fori_loop(0, CH // 16, grp, 0)

        @pl.when(valid)
        def _():
            pltpu.sync_copy(exco, den_sh.at[packs[b].at[1]], add=True)

    _run_pass(lambda it: sub + it * NSUB, P1_ITERS, False, _p1_compute)
    plsc.subcore_barrier()

    def _p2_compute(b, valid):
        pltpu.sync_copy(den_sh.at[packs[b].at[1]], dnrs[b])

        def grp(g, _):
            for h, (e_ids, ex) in enumerate(_edge_scores(packs[b], b, g)):
                den = plsc.load_gather(dnrs[b],
                                       [e_ids, jnp.full((16,), h, jnp.int32)])
                coeff = ex / (den + 1e-16)
                cfT[h, pl.ds(g * 16, 16)] = coeff
            return 0

        lax.fori_loop(0, CH // 16, grp, 0)

        def grp2(g, _):
            cvs = [cfT[h, pl.ds(g * 16, 16)] for h in range(H)]
            for eo in range(16):
                e = g * 16 + eo
                eo16 = jnp.full((16,), eo, jnp.int32)
                for h in range(H):
                    sp = cvs[h].at[eo16].get(mode="promise_in_bounds")
                    xrs[b][e, pl.ds(h * 16, 16)] = (
                        xrs[b][e, pl.ds(h * 16, 16)] * sp)
            return 0

        lax.fori_loop(0, CH // 16, grp2, 0)

        @pl.when(valid)
        def _():
            pltpu.sync_copy(xrs[b], out_sh.at[packs[b].at[1]], add=True)

    _run_pass(lambda it: wid + it * 32, P2_ITERS, True, _p2_compute)
    plsc.subcore_barrier()

    r0w = sub * 624
    pltpu.sync_copy(out_sh.at[pl.ds(r0w, 624)],
                    out_hbm.at[core, pl.ds(r0w, 624)])

    @pl.when(sub == NSUB - 1)
    def _tail():
        pltpu.sync_copy(out_sh.at[pl.ds(N - 16, 16)],
                        out_hbm.at[core, pl.ds(N - 16, 16)])


def kernel(x, edge_weight, W, att_l, att_r, W_res, edge_index):
    attl = att_l.reshape(H, C)
    attr_ = att_r.reshape(H, C)
    eyeH = jnp.eye(H, dtype=jnp.float32)
    A_l = (attl[:, :, None] * eyeH[:, None, :]).reshape(HC, H)
    A_r = (attr_[:, :, None] * eyeH[:, None, :]).reshape(HC, H)
    A = jnp.concatenate([A_l, A_r], axis=1)

    src2 = edge_index[0].reshape(NCHUNK, CH)
    dst2 = edge_index[1].reshape(NCHUNK, CH)
    ew2 = lax.bitcast_convert_type(edge_weight, jnp.int32).reshape(NCHUNK, CH)
    pack = jnp.stack([src2, dst2, ew2], axis=1)

    xp, alr = _stage1(x, W, A)
    out2 = _sc_edge(alr, xp, pack)
    return _stage3(out2[0], out2[1], x, W_res)


# speedup vs baseline: 84.7725x; 1.0844x over previous
import functools

import jax
import jax.numpy as jnp
from jax import lax
from jax.experimental import pallas as pl
from jax.experimental.pallas import tpu as pltpu
from jax.experimental.pallas import tpu_sc as plsc

N = 10000
E = 320000
D = 128
H = 8
C = 16
HC = H * C
CH = 128
NCHUNK = E // CH
NSUB = 16
ROWS_PER_SUB = N // NSUB
NB = 10
BLK = N // NB

P1_ITERS = 158
P2_ITERS = 80


def _stage1_body(x_ref, w_ref, a_ref, xp_ref, alr_ref):
    xp = jnp.dot(x_ref[...], w_ref[...], preferred_element_type=jnp.float32,
                 precision=lax.Precision.HIGHEST)
    xp_ref[...] = xp
    alr_ref[...] = jnp.dot(xp, a_ref[...], preferred_element_type=jnp.float32,
                           precision=lax.Precision.HIGHEST)


def _stage1(x, W, A):
    return pl.pallas_call(
        _stage1_body,
        grid=(NB,),
        in_specs=[pl.BlockSpec((BLK, D), lambda i: (i, 0)),
                  pl.BlockSpec((D, HC), lambda i: (0, 0)),
                  pl.BlockSpec((D, 2 * H), lambda i: (0, 0))],
        out_specs=[pl.BlockSpec((BLK, HC), lambda i: (i, 0)),
                   pl.BlockSpec((BLK, 2 * H), lambda i: (i, 0))],
        out_shape=[jax.ShapeDtypeStruct((N, HC), jnp.float32),
                   jax.ShapeDtypeStruct((N, 2 * H), jnp.float32)],
    )(x, W, A)


def _stage3_body(o0_ref, o1_ref, x_ref, wr_ref, out_ref):
    v = o0_ref[...] + o1_ref[...]
    v = jnp.where(v > 0, v, jnp.exp(jnp.minimum(v, 0.0)) - 1.0)
    out_ref[...] = v + jnp.dot(x_ref[...], wr_ref[...],
                               preferred_element_type=jnp.float32,
                               precision=lax.Precision.HIGHEST)


def _stage3(o0, o1, x, W_res):
    return pl.pallas_call(
        _stage3_body,
        grid=(NB,),
        in_specs=[pl.BlockSpec((BLK, HC), lambda i: (i, 0)),
                  pl.BlockSpec((BLK, HC), lambda i: (i, 0)),
                  pl.BlockSpec((BLK, D), lambda i: (i, 0)),
                  pl.BlockSpec((D, HC), lambda i: (0, 0))],
        out_specs=pl.BlockSpec((BLK, HC), lambda i: (i, 0)),
        out_shape=jax.ShapeDtypeStruct((N, HC), jnp.float32),
    )(o0, o1, x, W_res)


_sc_mesh = plsc.VectorSubcoreMesh(core_axis_name="c", subcore_axis_name="s")
_sc_params = pltpu.CompilerParams(needs_layout_passes=False,
                                  use_tc_tiling_on_sc=False)
_LANE = None


def _edge_scores(pack_b, srows, drows, lane, g):
    e_ids = g * 16 + lane
    ewv = plsc.bitcast(pack_b[2, pl.ds(g * 16, 16)], jnp.float32)
    res = []
    for h in range(H):
        al = plsc.load_gather(srows, [e_ids, jnp.full((16,), h, jnp.int32)])
        ar = plsc.load_gather(drows, [e_ids, jnp.full((16,), h + 8, jnp.int32)])
        t = ewv * (al + ar)
        t = jnp.where(t >= 0, t, 0.2 * t)
        res.append((e_ids, jnp.exp(t)))
    return res


def _clamped_chunk(c):
    return jnp.minimum(c, NCHUNK - 1), c < NCHUNK


@functools.partial(
    pl.kernel,
    out_type=jax.ShapeDtypeStruct((2, N, H), jnp.float32),
    mesh=_sc_mesh,
    compiler_params=_sc_params,
    scratch_types=[
        pltpu.VMEM((3, CH), jnp.int32),
        pltpu.VMEM((3, CH), jnp.int32),
        pltpu.VMEM((CH, 16), jnp.float32),
        pltpu.VMEM((CH, 16), jnp.float32),
        pltpu.VMEM((CH, 16), jnp.float32),
        pltpu.VMEM((CH, 16), jnp.float32),
        pltpu.VMEM((CH, H), jnp.float32),
        pltpu.VMEM((CH, H), jnp.float32),
        pltpu.VMEM_SHARED((N, H), jnp.float32),
        pltpu.SemaphoreType.DMA,
        pltpu.SemaphoreType.DMA,
    ],
)
def _sc_den(alr_hbm, pack_hbm, den_hbm,
            pack0, pack1, sr0, sr1, dr0, dr1, ex0, ex1, den_sh, semi, semg):
    core = lax.axis_index("c")
    sub = lax.axis_index("s")
    lane = lax.iota(jnp.int32, 16)
    z16 = jnp.zeros((16,), jnp.float32)
    packs = (pack0, pack1)
    srs = (sr0, sr1)
    drs = (dr0, dr1)
    excos = (ex0, ex1)

    def _zero_row(i, _):
        plsc.store_scatter(ex0, [jnp.full((16,), i, jnp.int32),
                                 jnp.bitwise_and(lane, 7)], z16)
        return 0

    lax.fori_loop(0, CH, _zero_row, 0)
    r0 = sub * ROWS_PER_SUB
    for k in range(5):
        pltpu.sync_copy(ex0.at[pl.ds(0, 125)],
                        den_sh.at[pl.ds(r0 + k * 125, 125)])
    plsc.subcore_barrier()

    def fire_idx(it, p):
        c, _ = _clamped_chunk(sub + it * NSUB)
        pltpu.async_copy(pack_hbm.at[c], packs[p], semi)

    def drain_idx(p):
        pltpu.make_async_copy(pack_hbm.at[0], packs[p], semi).wait()

    def fire_gathers(b):
        pltpu.async_copy(alr_hbm.at[packs[b].at[0]], srs[b], semg)
        pltpu.async_copy(alr_hbm.at[packs[b].at[1]], drs[b], semg)

    def drain_gathers(b):
        pltpu.make_async_copy(alr_hbm.at[packs[b].at[0]], srs[b], semg).wait()
        pltpu.make_async_copy(alr_hbm.at[packs[b].at[1]], drs[b], semg).wait()

    c0, _ = _clamped_chunk(sub)
    pltpu.sync_copy(pack_hbm.at[c0], pack0)
    fire_gathers(0)
    fire_idx(1, 1)

    def body(j, _):
        for b in (0, 1):
            it = 2 * j + b
            drain_gathers(b)
            drain_idx(b ^ 1)
            fire_gathers(b ^ 1)

            def grp(g, _):
                for h, (e_ids, ex) in enumerate(
                        _edge_scores(packs[b], srs[b], drs[b], lane, g)):
                    plsc.store_scatter(
                        excos[b], [e_ids, jnp.full((16,), h, jnp.int32)], ex)
                return 0

            lax.fori_loop(0, CH // 16, grp, 0)
            _, valid = _clamped_chunk(sub + it * NSUB)

            @pl.when(valid)
            def _():
                pltpu.sync_copy(excos[b], den_sh.at[packs[b].at[1]], add=True)

            fire_idx(it + 2, b)
        return 0

    lax.fori_loop(0, P1_ITERS // 2, body, 0)
    drain_gathers(0)
    drain_idx(1)
    plsc.subcore_barrier()

    r0w = sub * 624
    pltpu.sync_copy(den_sh.at[pl.ds(r0w, 624)],
                    den_hbm.at[core, pl.ds(r0w, 624)])

    @pl.when(sub == NSUB - 1)
    def _tail():
        pltpu.sync_copy(den_sh.at[pl.ds(N - 16, 16)],
                        den_hbm.at[core, pl.ds(N - 16, 16)])


@functools.partial(
    pl.kernel,
    out_type=jax.ShapeDtypeStruct((2, N, HC), jnp.float32),
    mesh=_sc_mesh,
    compiler_params=_sc_params,
    scratch_types=[
        pltpu.VMEM((3, CH), jnp.int32),
        pltpu.VMEM((3, CH), jnp.int32),
        pltpu.VMEM((1, CH), jnp.int32),
        pltpu.VMEM((1, CH), jnp.int32),
        pltpu.VMEM((CH, 16), jnp.float32),
        pltpu.VMEM((CH, 16), jnp.float32),
        pltpu.VMEM((CH, 16), jnp.float32),
        pltpu.VMEM((CH, 16), jnp.float32),
        pltpu.VMEM((CH, HC), jnp.float32),
        pltpu.VMEM((CH, HC), jnp.float32),
        pltpu.VMEM((CH, H), jnp.float32),
        pltpu.VMEM((CH, H), jnp.float32),
        pltpu.VMEM((H, CH), jnp.float32),
        pltpu.VMEM_SHARED((N, HC), jnp.float32),
        pltpu.SemaphoreType.DMA,
        pltpu.SemaphoreType.DMA,
        pltpu.SemaphoreType.DMA,
    ],
)
def _sc_out(alr_hbm, xp_hbm, pack_hbm, den_hbm, out_hbm,
            pack0, pack1, dstc0, dstc1, sr0, sr1, dr0, dr1, xr0, xr1,
            dnr0, dnr1, cfT, out_sh, semi, semg, semsc):
    core = lax.axis_index("c")
    sub = lax.axis_index("s")
    wid = sub * 2 + core
    lane = lax.iota(jnp.int32, 16)
    z16 = jnp.zeros((16,), jnp.float32)
    packs = (pack0, pack1)
    dstcs = (dstc0, dstc1)
    srs = (sr0, sr1)
    drs = (dr0, dr1)
    xrs = (xr0, xr1)
    dnrs = (dnr0, dnr1)

    def _zero_row(i, _):
        for j in range(HC // 16):
            xr0[i, pl.ds(16 * j, 16)] = z16
        return 0

    lax.fori_loop(0, CH, _zero_row, 0)
    r0 = sub * ROWS_PER_SUB
    for k in range(5):
        pltpu.sync_copy(xr0.at[pl.ds(0, 125)],
                        out_sh.at[pl.ds(r0 + k * 125, 125)])
    plsc.subcore_barrier()

    def fire_idx(it, p):
        c, _ = _clamped_chunk(wid + it * 32)
        pltpu.async_copy(pack_hbm.at[c], packs[p], semi)

    def drain_idx(p):
        pltpu.make_async_copy(pack_hbm.at[0], packs[p], semi).wait()

    def fire_gathers(b):
        pltpu.async_copy(alr_hbm.at[packs[b].at[0]], srs[b], semg)
        pltpu.async_copy(alr_hbm.at[packs[b].at[1]], drs[b], semg)
        pltpu.async_copy(xp_hbm.at[packs[b].at[0]], xrs[b], semg)
        pltpu.async_copy(den_hbm.at[core].at[packs[b].at[1]], dnrs[b], semg)

    def drain_gathers(b):
        pltpu.make_async_copy(alr_hbm.at[packs[b].at[0]], srs[b], semg).wait()
        pltpu.make_async_copy(alr_hbm.at[packs[b].at[1]], drs[b], semg).wait()
        pltpu.make_async_copy(xp_hbm.at[packs[b].at[0]], xrs[b], semg).wait()
        pltpu.make_async_copy(den_hbm.at[core].at[packs[b].at[1]], dnrs[b],
                              semg).wait()

    def fire_scatter(b):
        pltpu.async_copy(xrs[b], out_sh.at[dstcs[b].at[0]], semsc, add=True)

    def drain_scatter(b):
        pltpu.make_async_copy(xrs[b], out_sh.at[dstcs[b].at[0]], semsc).wait()

    c0, _ = _clamped_chunk(wid)
    pltpu.sync_copy(pack_hbm.at[c0], pack0)
    fire_gathers(0)
    fire_idx(1, 1)

    def body(j, _):
        for b in (0, 1):
            it = 2 * j + b
            drain_gathers(b)
            drain_idx(b ^ 1)
            _, pvalid = _clamped_chunk(wid + (it - 1) * 32)

            @pl.when((it >= 1) & pvalid)
            def _():
                drain_scatter(b ^ 1)

            fire_gathers(b ^ 1)
            for jj in range(CH // 16):
                dstcs[b][0, pl.ds(16 * jj, 16)] = (
                    packs[b][1, pl.ds(16 * jj, 16)])

            def grp(g, _):
                for h, (e_ids, ex) in enumerate(
                        _edge_scores(packs[b], srs[b], drs[b], lane, g)):
                    den = plsc.load_gather(
                        dnrs[b], [e_ids, jnp.full((16,), h, jnp.int32)])
                    cfT[h, pl.ds(g * 16, 16)] = ex / (den + 1e-16)
                return 0

            lax.fori_loop(0, CH // 16, grp, 0)

            def grp2(g, _):
                cvs = [cfT[h, pl.ds(g * 16, 16)] for h in range(H)]
                for eo in range(16):
                    e = g * 16 + eo
                    eo16 = jnp.full((16,), eo, jnp.int32)
                    for h in range(H):
                        sp = cvs[h].at[eo16].get(mode="promise_in_bounds")
                        xrs[b][e, pl.ds(h * 16, 16)] = (
                            xrs[b][e, pl.ds(h * 16, 16)] * sp)
                return 0

            lax.fori_loop(0, CH // 16, grp2, 0)
            _, valid = _clamped_chunk(wid + it * 32)

            @pl.when(valid)
            def _():
                fire_scatter(b)

            fire_idx(it + 2, b)
        return 0

    lax.fori_loop(0, P2_ITERS // 2, body, 0)
    drain_gathers(0)
    drain_idx(1)
    _, lvalid = _clamped_chunk(wid + (P2_ITERS - 1) * 32)

    @pl.when(lvalid)
    def _():
        drain_scatter((P2_ITERS - 1) & 1)

    plsc.subcore_barrier()

    r0w = sub * 624
    pltpu.sync_copy(out_sh.at[pl.ds(r0w, 624)],
                    out_hbm.at[core, pl.ds(r0w, 624)])

    @pl.when(sub == NSUB - 1)
    def _tail():
        pltpu.sync_copy(out_sh.at[pl.ds(N - 16, 16)],
                        out_hbm.at[core, pl.ds(N - 16, 16)])


def kernel(x, edge_weight, W, att_l, att_r, W_res, edge_index):
    attl = att_l.reshape(H, C)
    attr_ = att_r.reshape(H, C)
    eyeH = jnp.eye(H, dtype=jnp.float32)
    A_l = (attl[:, :, None] * eyeH[:, None, :]).reshape(HC, H)
    A_r = (attr_[:, :, None] * eyeH[:, None, :]).reshape(HC, H)
    A = jnp.concatenate([A_l, A_r], axis=1)

    src2 = edge_index[0].reshape(NCHUNK, CH)
    dst2 = edge_index[1].reshape(NCHUNK, CH)
    ew2 = lax.bitcast_convert_type(edge_weight, jnp.int32).reshape(NCHUNK, CH)
    pack = jnp.stack([src2, dst2, ew2], axis=1)

    xp, alr = _stage1(x, W, A)
    den2 = _sc_den(alr, pack)
    out2 = _sc_out(alr, xp, pack, den2)
    return _stage3(out2[0], out2[1], x, W_res)


# speedup vs baseline: 100.4443x; 1.1849x over previous
import functools

import jax
import jax.numpy as jnp
from jax import lax
from jax.experimental import pallas as pl
from jax.experimental.pallas import tpu as pltpu
from jax.experimental.pallas import tpu_sc as plsc

N = 10000
E = 320000
D = 128
H = 8
C = 16
HC = H * C
CH = 128
NCHUNK = E // CH
NSUB = 16
ROWS_PER_SUB = N // NSUB
NB = 10
BLK = N // NB

P_ITERS = 80


def _stage1_body(x_ref, w_ref, a_ref, xp_ref, alr_ref):
    xp = jnp.dot(x_ref[...], w_ref[...], preferred_element_type=jnp.float32,
                 precision=lax.Precision.HIGHEST)
    xp_ref[...] = xp
    alr_ref[...] = jnp.dot(xp, a_ref[...], preferred_element_type=jnp.float32,
                           precision=lax.Precision.HIGHEST)


def _stage1(x, W, A):
    return pl.pallas_call(
        _stage1_body,
        grid=(NB,),
        in_specs=[pl.BlockSpec((BLK, D), lambda i: (i, 0)),
                  pl.BlockSpec((D, HC), lambda i: (0, 0)),
                  pl.BlockSpec((D, 2 * H), lambda i: (0, 0))],
        out_specs=[pl.BlockSpec((BLK, HC), lambda i: (i, 0)),
                   pl.BlockSpec((BLK, 2 * H), lambda i: (i, 0))],
        out_shape=[jax.ShapeDtypeStruct((N, HC), jnp.float32),
                   jax.ShapeDtypeStruct((N, 2 * H), jnp.float32)],
    )(x, W, A)


def _stage3_body(o0_ref, o1_ref, x_ref, wr_ref, out_ref):
    v = o0_ref[...] + o1_ref[...]
    v = jnp.where(v > 0, v, jnp.exp(jnp.minimum(v, 0.0)) - 1.0)
    out_ref[...] = v + jnp.dot(x_ref[...], wr_ref[...],
                               preferred_element_type=jnp.float32,
                               precision=lax.Precision.HIGHEST)


def _stage3(o0, o1, x, W_res):
    return pl.pallas_call(
        _stage3_body,
        grid=(NB,),
        in_specs=[pl.BlockSpec((BLK, HC), lambda i: (i, 0)),
                  pl.BlockSpec((BLK, HC), lambda i: (i, 0)),
                  pl.BlockSpec((BLK, D), lambda i: (i, 0)),
                  pl.BlockSpec((D, HC), lambda i: (0, 0))],
        out_specs=pl.BlockSpec((BLK, HC), lambda i: (i, 0)),
        out_shape=jax.ShapeDtypeStruct((N, HC), jnp.float32),
    )(o0, o1, x, W_res)


_sc_mesh = plsc.VectorSubcoreMesh(core_axis_name="c", subcore_axis_name="s")
_sc_params = pltpu.CompilerParams(needs_layout_passes=False,
                                  use_tc_tiling_on_sc=False)
_LANE = None


def _edge_scores(pack_b, srows, drows, lane, g):
    e_ids = g * 16 + lane
    ewv = plsc.bitcast(pack_b[2, pl.ds(g * 16, 16)], jnp.float32)
    res = []
    for h in range(H):
        al = plsc.load_gather(srows, [e_ids, jnp.full((16,), h, jnp.int32)])
        ar = plsc.load_gather(drows, [e_ids, jnp.full((16,), h + 8, jnp.int32)])
        t = ewv * (al + ar)
        t = jnp.where(t >= 0, t, 0.2 * t)
        res.append((e_ids, jnp.exp(t)))
    return res


def _clamped_chunk(c):
    return jnp.minimum(c, NCHUNK - 1), c < NCHUNK


@functools.partial(
    pl.kernel,
    out_type=jax.ShapeDtypeStruct((2, N, H), jnp.float32),
    mesh=_sc_mesh,
    compiler_params=_sc_params,
    scratch_types=[
        pltpu.VMEM((3, CH), jnp.int32),
        pltpu.VMEM((3, CH), jnp.int32),
        pltpu.VMEM((CH, 16), jnp.float32),
        pltpu.VMEM((CH, 16), jnp.float32),
        pltpu.VMEM((CH, 16), jnp.float32),
        pltpu.VMEM((CH, 16), jnp.float32),
        pltpu.VMEM((CH, H), jnp.float32),
        pltpu.VMEM((CH, H), jnp.float32),
        pltpu.VMEM_SHARED((N, H), jnp.float32),
        pltpu.SemaphoreType.DMA,
        pltpu.SemaphoreType.DMA,
    ],
)
def _sc_den(alr_hbm, pack_hbm, den_hbm,
            pack0, pack1, sr0, sr1, dr0, dr1, ex0, ex1, den_sh, semi, semg):
    core = lax.axis_index("c")
    sub = lax.axis_index("s")
    wid = sub * 2 + core
    lane = lax.iota(jnp.int32, 16)
    z16 = jnp.zeros((16,), jnp.float32)
    packs = (pack0, pack1)
    srs = (sr0, sr1)
    drs = (dr0, dr1)
    excos = (ex0, ex1)

    def _zero_row(i, _):
        plsc.store_scatter(ex0, [jnp.full((16,), i, jnp.int32),
                                 jnp.bitwise_and(lane, 7)], z16)
        return 0

    lax.fori_loop(0, CH, _zero_row, 0)
    r0 = sub * ROWS_PER_SUB
    for k in range(5):
        pltpu.sync_copy(ex0.at[pl.ds(0, 125)],
                        den_sh.at[pl.ds(r0 + k * 125, 125)])
    plsc.subcore_barrier()

    def fire_idx(it, p):
        c, _ = _clamped_chunk(wid + it * 32)
        pltpu.async_copy(pack_hbm.at[c], packs[p], semi)

    def drain_idx(p):
        pltpu.make_async_copy(pack_hbm.at[0], packs[p], semi).wait()

    def fire_gathers(b):
        pltpu.async_copy(alr_hbm.at[packs[b].at[0]], srs[b], semg)
        pltpu.async_copy(alr_hbm.at[packs[b].at[1]], drs[b], semg)

    def drain_gathers(b):
        pltpu.make_async_copy(alr_hbm.at[packs[b].at[0]], srs[b], semg).wait()
        pltpu.make_async_copy(alr_hbm.at[packs[b].at[1]], drs[b], semg).wait()

    c0, _ = _clamped_chunk(wid)
    pltpu.sync_copy(pack_hbm.at[c0], pack0)
    fire_gathers(0)
    fire_idx(1, 1)

    def body(j, _):
        for b in (0, 1):
            it = 2 * j + b
            drain_gathers(b)
            drain_idx(b ^ 1)
            fire_gathers(b ^ 1)

            def grp(g, _):
                for h, (e_ids, ex) in enumerate(
                        _edge_scores(packs[b], srs[b], drs[b], lane, g)):
                    plsc.store_scatter(
                        excos[b], [e_ids, jnp.full((16,), h, jnp.int32)], ex)
                return 0

            lax.fori_loop(0, CH // 16, grp, 0)
            _, valid = _clamped_chunk(wid + it * 32)

            @pl.when(valid)
            def _():
                pltpu.sync_copy(excos[b], den_sh.at[packs[b].at[1]], add=True)

            fire_idx(it + 2, b)
        return 0

    lax.fori_loop(0, P_ITERS // 2, body, 0)
    drain_gathers(0)
    drain_idx(1)
    plsc.subcore_barrier()

    r0w = sub * 624
    pltpu.sync_copy(den_sh.at[pl.ds(r0w, 624)],
                    den_hbm.at[core, pl.ds(r0w, 624)])

    @pl.when(sub == NSUB - 1)
    def _tail():
        pltpu.sync_copy(den_sh.at[pl.ds(N - 16, 16)],
                        den_hbm.at[core, pl.ds(N - 16, 16)])


@functools.partial(
    pl.kernel,
    out_type=jax.ShapeDtypeStruct((2, N, HC), jnp.float32),
    mesh=_sc_mesh,
    compiler_params=_sc_params,
    scratch_types=[
        pltpu.VMEM((3, CH), jnp.int32),
        pltpu.VMEM((3, CH), jnp.int32),
        pltpu.VMEM((1, CH), jnp.int32),
        pltpu.VMEM((1, CH), jnp.int32),
        pltpu.VMEM((CH, 16), jnp.float32),
        pltpu.VMEM((CH, 16), jnp.float32),
        pltpu.VMEM((CH, 16), jnp.float32),
        pltpu.VMEM((CH, 16), jnp.float32),
        pltpu.VMEM((CH, HC), jnp.float32),
        pltpu.VMEM((CH, HC), jnp.float32),
        pltpu.VMEM((CH, H), jnp.float32),
        pltpu.VMEM((CH, H), jnp.float32),
        pltpu.VMEM((CH, H), jnp.float32),
        pltpu.VMEM((CH, H), jnp.float32),
        pltpu.VMEM((H, CH), jnp.float32),
        pltpu.VMEM_SHARED((N, HC), jnp.float32),
        pltpu.SemaphoreType.DMA,
        pltpu.SemaphoreType.DMA,
        pltpu.SemaphoreType.DMA,
    ],
)
def _sc_out(alr_hbm, xp_hbm, pack_hbm, den_hbm, out_hbm,
            pack0, pack1, dstc0, dstc1, sr0, sr1, dr0, dr1, xr0, xr1,
            dnra0, dnra1, dnrb0, dnrb1, cfT, out_sh, semi, semg, semsc):
    core = lax.axis_index("c")
    sub = lax.axis_index("s")
    wid = sub * 2 + core
    lane = lax.iota(jnp.int32, 16)
    z16 = jnp.zeros((16,), jnp.float32)
    packs = (pack0, pack1)
    dstcs = (dstc0, dstc1)
    srs = (sr0, sr1)
    drs = (dr0, dr1)
    xrs = (xr0, xr1)
    dnras = (dnra0, dnra1)
    dnrbs = (dnrb0, dnrb1)

    def _zero_row(i, _):
        for j in range(HC // 16):
            xr0[i, pl.ds(16 * j, 16)] = z16
        return 0

    lax.fori_loop(0, CH, _zero_row, 0)
    r0 = sub * ROWS_PER_SUB
    for k in range(5):
        pltpu.sync_copy(xr0.at[pl.ds(0, 125)],
                        out_sh.at[pl.ds(r0 + k * 125, 125)])
    plsc.subcore_barrier()

    def fire_idx(it, p):
        c, _ = _clamped_chunk(wid + it * 32)
        pltpu.async_copy(pack_hbm.at[c], packs[p], semi)

    def drain_idx(p):
        pltpu.make_async_copy(pack_hbm.at[0], packs[p], semi).wait()

    def fire_gathers(b):
        pltpu.async_copy(alr_hbm.at[packs[b].at[0]], srs[b], semg)
        pltpu.async_copy(alr_hbm.at[packs[b].at[1]], drs[b], semg)
        pltpu.async_copy(xp_hbm.at[packs[b].at[0]], xrs[b], semg)
        pltpu.async_copy(den_hbm.at[0].at[packs[b].at[1]], dnras[b], semg)
        pltpu.async_copy(den_hbm.at[1].at[packs[b].at[1]], dnrbs[b], semg)

    def drain_gathers(b):
        pltpu.make_async_copy(alr_hbm.at[packs[b].at[0]], srs[b], semg).wait()
        pltpu.make_async_copy(alr_hbm.at[packs[b].at[1]], drs[b], semg).wait()
        pltpu.make_async_copy(xp_hbm.at[packs[b].at[0]], xrs[b], semg).wait()
        pltpu.make_async_copy(den_hbm.at[0].at[packs[b].at[1]], dnras[b],
                              semg).wait()
        pltpu.make_async_copy(den_hbm.at[1].at[packs[b].at[1]], dnrbs[b],
                              semg).wait()

    def fire_scatter(b):
        pltpu.async_copy(xrs[b], out_sh.at[dstcs[b].at[0]], semsc, add=True)

    def drain_scatter(b):
        pltpu.make_async_copy(xrs[b], out_sh.at[dstcs[b].at[0]], semsc).wait()

    c0, _ = _clamped_chunk(wid)
    pltpu.sync_copy(pack_hbm.at[c0], pack0)
    fire_gathers(0)
    fire_idx(1, 1)

    def body(j, _):
        for b in (0, 1):
            it = 2 * j + b
            drain_gathers(b)
            drain_idx(b ^ 1)
            _, pvalid = _clamped_chunk(wid + (it - 1) * 32)

            @pl.when((it >= 1) & pvalid)
            def _():
                drain_scatter(b ^ 1)

            fire_gathers(b ^ 1)
            for jj in range(CH // 16):
                dstcs[b][0, pl.ds(16 * jj, 16)] = (
                    packs[b][1, pl.ds(16 * jj, 16)])

            def grp(g, _):
                for h, (e_ids, ex) in enumerate(
                        _edge_scores(packs[b], srs[b], drs[b], lane, g)):
                    h16 = jnp.full((16,), h, jnp.int32)
                    den = (plsc.load_gather(dnras[b], [e_ids, h16]) +
                           plsc.load_gather(dnrbs[b], [e_ids, h16]))
                    cfT[h, pl.ds(g * 16, 16)] = ex / (den + 1e-16)
                return 0

            lax.fori_loop(0, CH // 16, grp, 0)

            def grp2(g, _):
                cvs = [cfT[h, pl.ds(g * 16, 16)] for h in range(H)]
                for eo in range(16):
                    e = g * 16 + eo
                    eo16 = jnp.full((16,), eo, jnp.int32)
                    for h in range(H):
                        sp = cvs[h].at[eo16].get(mode="promise_in_bounds")
                        xrs[b][e, pl.ds(h * 16, 16)] = (
                            xrs[b][e, pl.ds(h * 16, 16)] * sp)
                return 0

            lax.fori_loop(0, CH // 16, grp2, 0)
            _, valid = _clamped_chunk(wid + it * 32)

            @pl.when(valid)
            def _():
                fire_scatter(b)

            fire_idx(it + 2, b)
        return 0

    lax.fori_loop(0, P_ITERS // 2, body, 0)
    drain_gathers(0)
    drain_idx(1)
    _, lvalid = _clamped_chunk(wid + (P_ITERS - 1) * 32)

    @pl.when(lvalid)
    def _():
        drain_scatter((P_ITERS - 1) & 1)

    plsc.subcore_barrier()

    r0w = sub * 624
    pltpu.sync_copy(out_sh.at[pl.ds(r0w, 624)],
                    out_hbm.at[core, pl.ds(r0w, 624)])

    @pl.when(sub == NSUB - 1)
    def _tail():
        pltpu.sync_copy(out_sh.at[pl.ds(N - 16, 16)],
                        out_hbm.at[core, pl.ds(N - 16, 16)])


def kernel(x, edge_weight, W, att_l, att_r, W_res, edge_index):
    attl = att_l.reshape(H, C)
    attr_ = att_r.reshape(H, C)
    eyeH = jnp.eye(H, dtype=jnp.float32)
    A_l = (attl[:, :, None] * eyeH[:, None, :]).reshape(HC, H)
    A_r = (attr_[:, :, None] * eyeH[:, None, :]).reshape(HC, H)
    A = jnp.concatenate([A_l, A_r], axis=1)

    src2 = edge_index[0].reshape(NCHUNK, CH)
    dst2 = edge_index[1].reshape(NCHUNK, CH)
    ew2 = lax.bitcast_convert_type(edge_weight, jnp.int32).reshape(NCHUNK, CH)
    pack = jnp.stack([src2, dst2, ew2], axis=1)

    xp, alr = _stage1(x, W, A)
    den2 = _sc_den(alr, pack)
    out2 = _sc_out(alr, xp, pack, den2)
    return _stage3(out2[0], out2[1], x, W_res)


# speedup vs baseline: 103.3690x; 1.0291x over previous
import functools

import jax
import jax.numpy as jnp
from jax import lax
from jax.experimental import pallas as pl
from jax.experimental.pallas import tpu as pltpu
from jax.experimental.pallas import tpu_sc as plsc

N = 10000
E = 320000
D = 128
H = 8
C = 16
HC = H * C
CH = 128
NCHUNK = E // CH
NSUB = 16
ROWS_PER_SUB = N // NSUB
NB = 10
BLK = N // NB

P_ITERS = 80


def _stage1_body(x_ref, w_ref, a_ref, xp_ref, alr_ref):
    xp = jnp.dot(x_ref[...], w_ref[...], preferred_element_type=jnp.float32,
                 precision=lax.Precision.HIGHEST)
    xp_ref[...] = xp
    alr_ref[...] = jnp.dot(xp, a_ref[...], preferred_element_type=jnp.float32,
                           precision=lax.Precision.HIGHEST)


def _stage1(x, W, A):
    return pl.pallas_call(
        _stage1_body,
        grid=(NB,),
        in_specs=[pl.BlockSpec((BLK, D), lambda i: (i, 0)),
                  pl.BlockSpec((D, HC), lambda i: (0, 0)),
                  pl.BlockSpec((D, 2 * H), lambda i: (0, 0))],
        out_specs=[pl.BlockSpec((BLK, HC), lambda i: (i, 0)),
                   pl.BlockSpec((BLK, 2 * H), lambda i: (i, 0))],
        out_shape=[jax.ShapeDtypeStruct((N, HC), jnp.float32),
                   jax.ShapeDtypeStruct((N, 2 * H), jnp.float32)],
    )(x, W, A)


def _stage3_body(o0_ref, o1_ref, x_ref, wr_ref, out_ref):
    v = o0_ref[...] + o1_ref[...]
    v = jnp.where(v > 0, v, jnp.exp(jnp.minimum(v, 0.0)) - 1.0)
    out_ref[...] = v + jnp.dot(x_ref[...], wr_ref[...],
                               preferred_element_type=jnp.float32,
                               precision=lax.Precision.HIGHEST)


def _stage3(o0, o1, x, W_res):
    return pl.pallas_call(
        _stage3_body,
        grid=(NB,),
        in_specs=[pl.BlockSpec((BLK, HC), lambda i: (i, 0)),
                  pl.BlockSpec((BLK, HC), lambda i: (i, 0)),
                  pl.BlockSpec((BLK, D), lambda i: (i, 0)),
                  pl.BlockSpec((D, HC), lambda i: (0, 0))],
        out_specs=pl.BlockSpec((BLK, HC), lambda i: (i, 0)),
        out_shape=jax.ShapeDtypeStruct((N, HC), jnp.float32),
    )(o0, o1, x, W_res)


_sc_mesh = plsc.VectorSubcoreMesh(core_axis_name="c", subcore_axis_name="s")
_sc_params = pltpu.CompilerParams(needs_layout_passes=False,
                                  use_tc_tiling_on_sc=False)
_LANE = None


def _edge_scores(pack_b, srows, drows, lane, g):
    e_ids = g * 16 + lane
    ewv = plsc.bitcast(pack_b[2, pl.ds(g * 16, 16)], jnp.float32)
    res = []
    for h in range(H):
        al = plsc.load_gather(srows, [e_ids, jnp.full((16,), h, jnp.int32)])
        ar = plsc.load_gather(drows, [e_ids, jnp.full((16,), h + 8, jnp.int32)])
        t = ewv * (al + ar)
        t = jnp.where(t >= 0, t, 0.2 * t)
        res.append((e_ids, jnp.exp(t)))
    return res


def _clamped_chunk(c):
    return jnp.minimum(c, NCHUNK - 1), c < NCHUNK


@functools.partial(
    pl.kernel,
    out_type=[jax.ShapeDtypeStruct((2, N, H), jnp.float32),
              jax.ShapeDtypeStruct((NCHUNK, CH, H), jnp.float32)],
    mesh=_sc_mesh,
    compiler_params=_sc_params,
    scratch_types=[
        pltpu.VMEM((3, CH), jnp.int32),
        pltpu.VMEM((3, CH), jnp.int32),
        pltpu.VMEM((CH, 16), jnp.float32),
        pltpu.VMEM((CH, 16), jnp.float32),
        pltpu.VMEM((CH, 16), jnp.float32),
        pltpu.VMEM((CH, 16), jnp.float32),
        pltpu.VMEM((CH, H), jnp.float32),
        pltpu.VMEM((CH, H), jnp.float32),
        pltpu.VMEM_SHARED((N, H), jnp.float32),
        pltpu.SemaphoreType.DMA,
        pltpu.SemaphoreType.DMA,
        pltpu.SemaphoreType.DMA,
    ],
)
def _sc_den(alr_hbm, pack_hbm, den_hbm, ex_hbm,
            pack0, pack1, sr0, sr1, dr0, dr1, ex0, ex1, den_sh,
            semi, semg, semx):
    core = lax.axis_index("c")
    sub = lax.axis_index("s")
    wid = sub * 2 + core
    lane = lax.iota(jnp.int32, 16)
    z16 = jnp.zeros((16,), jnp.float32)
    packs = (pack0, pack1)
    srs = (sr0, sr1)
    drs = (dr0, dr1)
    excos = (ex0, ex1)

    def _zero_row(i, _):
        plsc.store_scatter(ex0, [jnp.full((16,), i, jnp.int32),
                                 jnp.bitwise_and(lane, 7)], z16)
        return 0

    lax.fori_loop(0, CH, _zero_row, 0)
    r0 = sub * ROWS_PER_SUB
    for k in range(5):
        pltpu.sync_copy(ex0.at[pl.ds(0, 125)],
                        den_sh.at[pl.ds(r0 + k * 125, 125)])
    plsc.subcore_barrier()

    def fire_idx(it, p):
        c, _ = _clamped_chunk(wid + it * 32)
        pltpu.async_copy(pack_hbm.at[c], packs[p], semi)

    def drain_idx(p):
        pltpu.make_async_copy(pack_hbm.at[0], packs[p], semi).wait()

    def fire_gathers(b):
        pltpu.async_copy(alr_hbm.at[packs[b].at[0]], srs[b], semg)
        pltpu.async_copy(alr_hbm.at[packs[b].at[1]], drs[b], semg)

    def drain_gathers(b):
        pltpu.make_async_copy(alr_hbm.at[packs[b].at[0]], srs[b], semg).wait()
        pltpu.make_async_copy(alr_hbm.at[packs[b].at[1]], drs[b], semg).wait()

    c0, _ = _clamped_chunk(wid)
    pltpu.sync_copy(pack_hbm.at[c0], pack0)
    fire_gathers(0)
    fire_idx(1, 1)

    def body(j, _):
        for b in (0, 1):
            it = 2 * j + b
            drain_gathers(b)
            drain_idx(b ^ 1)
            _, pvalid = _clamped_chunk(wid + (it - 1) * 32)

            @pl.when((it >= 1) & pvalid)
            def _():
                cp, _ = _clamped_chunk(wid + (it - 1) * 32)
                pltpu.make_async_copy(excos[b ^ 1], ex_hbm.at[cp],
                                      semx).wait()

            fire_gathers(b ^ 1)

            def grp(g, _):
                for h, (e_ids, ex) in enumerate(
                        _edge_scores(packs[b], srs[b], drs[b], lane, g)):
                    plsc.store_scatter(
                        excos[b], [e_ids, jnp.full((16,), h, jnp.int32)], ex)
                return 0

            lax.fori_loop(0, CH // 16, grp, 0)
            c, valid = _clamped_chunk(wid + it * 32)

            @pl.when(valid)
            def _():
                pltpu.sync_copy(excos[b], den_sh.at[packs[b].at[1]], add=True)
                pltpu.async_copy(excos[b], ex_hbm.at[c], semx)

            fire_idx(it + 2, b)
        return 0

    lax.fori_loop(0, P_ITERS // 2, body, 0)
    drain_gathers(0)
    drain_idx(1)
    _, lvalid = _clamped_chunk(wid + (P_ITERS - 1) * 32)

    @pl.when(lvalid)
    def _():
        cl, _ = _clamped_chunk(wid + (P_ITERS - 1) * 32)
        pltpu.make_async_copy(excos[(P_ITERS - 1) & 1], ex_hbm.at[cl],
                              semx).wait()

    plsc.subcore_barrier()

    r0w = sub * 624
    pltpu.sync_copy(den_sh.at[pl.ds(r0w, 624)],
                    den_hbm.at[core, pl.ds(r0w, 624)])

    @pl.when(sub == NSUB - 1)
    def _tail():
        pltpu.sync_copy(den_sh.at[pl.ds(N - 16, 16)],
                        den_hbm.at[core, pl.ds(N - 16, 16)])


@functools.partial(
    pl.kernel,
    out_type=jax.ShapeDtypeStruct((2, N, HC), jnp.float32),
    mesh=_sc_mesh,
    compiler_params=_sc_params,
    scratch_types=[
        pltpu.VMEM((3, CH), jnp.int32),
        pltpu.VMEM((3, CH), jnp.int32),
        pltpu.VMEM((1, CH), jnp.int32),
        pltpu.VMEM((1, CH), jnp.int32),
        pltpu.VMEM((CH, H), jnp.float32),
        pltpu.VMEM((CH, H), jnp.float32),
        pltpu.VMEM((CH, HC), jnp.float32),
        pltpu.VMEM((CH, HC), jnp.float32),
        pltpu.VMEM((CH, H), jnp.float32),
        pltpu.VMEM((CH, H), jnp.float32),
        pltpu.VMEM((CH, H), jnp.float32),
        pltpu.VMEM((CH, H), jnp.float32),
        pltpu.VMEM((H, CH), jnp.float32),
        pltpu.VMEM_SHARED((N, HC), jnp.float32),
        pltpu.SemaphoreType.DMA,
        pltpu.SemaphoreType.DMA,
        pltpu.SemaphoreType.DMA,
    ],
)
def _sc_out(xp_hbm, pack_hbm, den_hbm, ex_hbm, out_hbm,
            pack0, pack1, dstc0, dstc1, exr0, exr1, xr0, xr1,
            dnra0, dnra1, dnrb0, dnrb1, cfT, out_sh, semi, semg, semsc):
    core = lax.axis_index("c")
    sub = lax.axis_index("s")
    wid = sub * 2 + core
    lane = lax.iota(jnp.int32, 16)
    z16 = jnp.zeros((16,), jnp.float32)
    packs = (pack0, pack1)
    dstcs = (dstc0, dstc1)
    exrs = (exr0, exr1)
    xrs = (xr0, xr1)
    dnras = (dnra0, dnra1)
    dnrbs = (dnrb0, dnrb1)

    def _zero_row(i, _):
        for j in range(HC // 16):
            xr0[i, pl.ds(16 * j, 16)] = z16
        return 0

    lax.fori_loop(0, CH, _zero_row, 0)
    r0 = sub * ROWS_PER_SUB
    for k in range(5):
        pltpu.sync_copy(xr0.at[pl.ds(0, 125)],
                        out_sh.at[pl.ds(r0 + k * 125, 125)])
    plsc.subcore_barrier()

    def fire_idx(it, p):
        c, _ = _clamped_chunk(wid + it * 32)
        pltpu.async_copy(pack_hbm.at[c], packs[p], semi)

    def drain_idx(p):
        pltpu.make_async_copy(pack_hbm.at[0], packs[p], semi).wait()

    def fire_gathers(b, it):
        c, _ = _clamped_chunk(wid + it * 32)
        pltpu.async_copy(ex_hbm.at[c], exrs[b], semg)
        pltpu.async_copy(xp_hbm.at[packs[b].at[0]], xrs[b], semg)
        pltpu.async_copy(den_hbm.at[0].at[packs[b].at[1]], dnras[b], semg)
        pltpu.async_copy(den_hbm.at[1].at[packs[b].at[1]], dnrbs[b], semg)

    def drain_gathers(b):
        pltpu.make_async_copy(ex_hbm.at[0], exrs[b], semg).wait()
        pltpu.make_async_copy(xp_hbm.at[packs[b].at[0]], xrs[b], semg).wait()
        pltpu.make_async_copy(den_hbm.at[0].at[packs[b].at[1]], dnras[b],
                              semg).wait()
        pltpu.make_async_copy(den_hbm.at[1].at[packs[b].at[1]], dnrbs[b],
                              semg).wait()

    def fire_scatter(b):
        pltpu.async_copy(xrs[b], out_sh.at[dstcs[b].at[0]], semsc, add=True)

    def drain_scatter(b):
        pltpu.make_async_copy(xrs[b], out_sh.at[dstcs[b].at[0]], semsc).wait()

    c0, _ = _clamped_chunk(wid)
    pltpu.sync_copy(pack_hbm.at[c0], pack0)
    fire_gathers(0, 0)
    fire_idx(1, 1)

    def body(j, _):
        for b in (0, 1):
            it = 2 * j + b
            drain_gathers(b)
            drain_idx(b ^ 1)
            _, pvalid = _clamped_chunk(wid + (it - 1) * 32)

            @pl.when((it >= 1) & pvalid)
            def _():
                drain_scatter(b ^ 1)

            fire_gathers(b ^ 1, it + 1)
            for jj in range(CH // 16):
                dstcs[b][0, pl.ds(16 * jj, 16)] = (
                    packs[b][1, pl.ds(16 * jj, 16)])

            def grp(g, _):
                e_ids = g * 16 + lane
                for h in range(H):
                    h16 = jnp.full((16,), h, jnp.int32)
                    ex = plsc.load_gather(exrs[b], [e_ids, h16])
                    den = (plsc.load_gather(dnras[b], [e_ids, h16]) +
                           plsc.load_gather(dnrbs[b], [e_ids, h16]))
                    cfT[h, pl.ds(g * 16, 16)] = ex / (den + 1e-16)
                return 0

            lax.fori_loop(0, CH // 16, grp, 0)

            def grp2(g, _):
                cvs = [cfT[h, pl.ds(g * 16, 16)] for h in range(H)]
                for eo in range(16):
                    e = g * 16 + eo
                    eo16 = jnp.full((16,), eo, jnp.int32)
                    for h in range(H):
                        sp = cvs[h].at[eo16].get(mode="promise_in_bounds")
                        xrs[b][e, pl.ds(h * 16, 16)] = (
                            xrs[b][e, pl.ds(h * 16, 16)] * sp)
                return 0

            lax.fori_loop(0, CH // 16, grp2, 0)
            _, valid = _clamped_chunk(wid + it * 32)

            @pl.when(valid)
            def _():
                fire_scatter(b)

            fire_idx(it + 2, b)
        return 0

    lax.fori_loop(0, P_ITERS // 2, body, 0)
    drain_gathers(0)
    drain_idx(1)
    _, lvalid = _clamped_chunk(wid + (P_ITERS - 1) * 32)

    @pl.when(lvalid)
    def _():
        drain_scatter((P_ITERS - 1) & 1)

    plsc.subcore_barrier()

    r0w = sub * 624
    pltpu.sync_copy(out_sh.at[pl.ds(r0w, 624)],
                    out_hbm.at[core, pl.ds(r0w, 624)])

    @pl.when(sub == NSUB - 1)
    def _tail():
        pltpu.sync_copy(out_sh.at[pl.ds(N - 16, 16)],
                        out_hbm.at[core, pl.ds(N - 16, 16)])


def kernel(x, edge_weight, W, att_l, att_r, W_res, edge_index):
    attl = att_l.reshape(H, C)
    attr_ = att_r.reshape(H, C)
    eyeH = jnp.eye(H, dtype=jnp.float32)
    A_l = (attl[:, :, None] * eyeH[:, None, :]).reshape(HC, H)
    A_r = (attr_[:, :, None] * eyeH[:, None, :]).reshape(HC, H)
    A = jnp.concatenate([A_l, A_r], axis=1)

    src2 = edge_index[0].reshape(NCHUNK, CH)
    dst2 = edge_index[1].reshape(NCHUNK, CH)
    ew2 = lax.bitcast_convert_type(edge_weight, jnp.int32).reshape(NCHUNK, CH)
    pack = jnp.stack([src2, dst2, ew2], axis=1)

    xp, alr = _stage1(x, W, A)
    den2, ex_all = _sc_den(alr, pack)
    out2 = _sc_out(xp, pack, den2, ex_all)
    return _stage3(out2[0], out2[1], x, W_res)
